# baseline (speedup 1.0000x reference)
import jax
import jax.numpy as jnp
from jax.experimental import pallas as pl
from jax.experimental.pallas import tpu as pltpu

N = 10000
NFEAT = 256
NOUT = 256
BK = 512
NK = (N + BK - 1) // BK
KREM = N - (NK - 1) * BK


def _stripe_dot(adj_ref, x_ref, w_ref, kslice):
    s = jax.lax.dot_general(
        x_ref[...], w_ref[...],
        dimension_numbers=(((1,), (0,)), ((), ())),
        precision=jax.lax.Precision.DEFAULT,
        preferred_element_type=jnp.float32,
    ).astype(jnp.bfloat16)
    return jax.lax.dot_general(
        adj_ref[:, kslice].astype(jnp.bfloat16), s[kslice, :],
        dimension_numbers=(((1,), (0,)), ((), ())),
        preferred_element_type=jnp.float32,
    )


def _gcn_kernel(adj_ref, x_ref, w_ref, b_ref, o_ref):
    k = pl.program_id(0)

    @pl.when(k == 0)
    def _():
        o_ref[...] = _stripe_dot(adj_ref, x_ref, w_ref, slice(None))

    @pl.when(jnp.logical_and(k > 0, k < NK - 1))
    def _():
        o_ref[...] += _stripe_dot(adj_ref, x_ref, w_ref, slice(None))

    @pl.when(k == NK - 1)
    def _():
        acc = o_ref[...] + _stripe_dot(adj_ref, x_ref, w_ref,
                                       slice(0, KREM))
        o_ref[...] = jnp.maximum(acc + b_ref[...], 0.0)


@jax.jit
def kernel(x, adj, W, b):
    b2 = b.reshape(1, NOUT)
    return pl.pallas_call(
        _gcn_kernel,
        grid=(NK,),
        out_shape=jax.ShapeDtypeStruct((N, NOUT), jnp.float32),
        in_specs=[
            pl.BlockSpec((N, BK), lambda k: (0, k)),
            pl.BlockSpec((BK, NFEAT), lambda k: (k, 0)),
            pl.BlockSpec((NFEAT, NOUT), lambda k: (0, 0)),
            pl.BlockSpec((1, NOUT), lambda k: (0, 0)),
        ],
        out_specs=pl.BlockSpec((N, NOUT), lambda k: (0, 0)),
        compiler_params=pltpu.CompilerParams(
            dimension_semantics=("arbitrary",),
            vmem_limit_bytes=58 * 1024 * 1024,
        ),
    )(adj, x, W, b2)
